# Initial kernel scaffold; baseline (speedup 1.0000x reference)
import jax
import jax.numpy as jnp
from jax.experimental import pallas as pl


def kernel(x, edge_index, edge_weight, node_graph_index, gin_W1, gin_b1, gin_W2, gin_b2, bn_gamma, bn_beta, mlp_w1, mlp_b1, mlp_w2, mlp_b2):
    raise NotImplementedError("write your pallas kernel here")


# speedup vs baseline: 3.9082x; 3.9082x over previous
import functools
import math

import jax
import jax.numpy as jnp
from jax import lax
from jax.experimental import pallas as pl
from jax.experimental.pallas import tpu as pltpu
from jax.experimental.pallas import tpu_sc as plsc

NC = 2
NS = 16
NW = NC * NS


def _make_sc_aggregate(n_pad, n_edges, d):
    assert n_edges % NW == 0
    epw = n_edges // NW
    K = 80
    assert epw % K == 0
    n_chunks = epw // K
    RB = 128
    assert n_pad % (NS * RB) == 0
    rpt = n_pad // NS
    n_rb = rpt // RB
    nvr = d // 16

    mesh = plsc.VectorSubcoreMesh(
        core_axis_name="c", subcore_axis_name="s",
        num_cores=NC, num_subcores=NS)

    @functools.partial(
        pl.kernel,
        out_type=jax.ShapeDtypeStruct((NC, n_pad, d), jnp.float32),
        mesh=mesh,
        scratch_types=[
            pltpu.VMEM((K,), jnp.int32),
            pltpu.VMEM((K,), jnp.int32),
            pltpu.VMEM((K,), jnp.float32),
            pltpu.VMEM((K, d), jnp.float32),
            pltpu.VMEM((RB, d), jnp.float32),
            pltpu.VMEM_SHARED((n_pad, d), jnp.float32),
            pltpu.SemaphoreType.DMA,
        ],
    )
    def sc_aggregate(h_hbm, src_hbm, dst_hbm, w_hbm, out_hbm,
                     src_v, dst_v, w_v, rows_v, bounce_v, aggr_sh, sem):
        c = lax.axis_index("c")
        s = lax.axis_index("s")
        wid = c * NS + s

        def zrow(i, carry):
            for r in range(nvr):
                bounce_v[i, pl.ds(r * 16, 16)] = jnp.zeros((16,), jnp.float32)
            return carry
        lax.fori_loop(0, RB, zrow, 0)
        row0 = s * rpt
        for b in range(n_rb):
            pltpu.sync_copy(bounce_v, aggr_sh.at[pl.ds(row0 + b * RB, RB)])
        plsc.subcore_barrier()

        ebase = wid * epw

        def chunk(ci, carry):
            base = ebase + ci * K
            pltpu.sync_copy(src_hbm.at[pl.ds(base, K)], src_v)
            pltpu.sync_copy(dst_hbm.at[pl.ds(base, K)], dst_v)
            pltpu.sync_copy(w_hbm.at[pl.ds(base, K)], w_v)
            pltpu.async_copy(h_hbm.at[src_v], rows_v, sem).wait()

            def group(gi, gcarry):
                wchunk = w_v[pl.ds(gi * 16, 16)]
                for j in range(16):
                    wb = lax.gather(
                        wchunk, jnp.full((16, 1), j, jnp.int32),
                        dimension_numbers=lax.GatherDimensionNumbers(
                            offset_dims=(), collapsed_slice_dims=(0,),
                            start_index_map=(0,)),
                        slice_sizes=(1,),
                        mode=lax.GatherScatterMode.PROMISE_IN_BOUNDS)
                    e = gi * 16 + j
                    for r in range(nvr):
                        sl = pl.ds(r * 16, 16)
                        rows_v[e, sl] = rows_v[e, sl] * wb
                return gcarry
            lax.fori_loop(0, K // 16, group, 0)

            pltpu.sync_copy(rows_v, aggr_sh.at[dst_v], add=True)
            return carry
        lax.fori_loop(0, n_chunks, chunk, 0)

        plsc.subcore_barrier()

        for b in range(n_rb):
            r0 = row0 + b * RB
            pltpu.sync_copy(aggr_sh.at[pl.ds(r0, RB)], bounce_v)
            pltpu.sync_copy(bounce_v, out_hbm.at[c, pl.ds(r0, RB)])

    return sc_aggregate


def _tc_layer_body(t_ref, a_ref, w1_ref, b1_ref, w2_ref, b2_ref,
                   sp_ref, bp_ref, sn_ref, bn_ref, ngi_ref,
                   tn_ref, pool_ref, *, n_graphs):
    i = pl.program_id(0)
    u = t_ref[...] + a_ref[0] + a_ref[1]
    z = jnp.dot(u, w1_ref[...], preferred_element_type=jnp.float32) + b1_ref[...]
    z = jnp.maximum(z, 0.0)
    g = jnp.dot(z, w2_ref[...], preferred_element_type=jnp.float32) + b2_ref[...]
    tn_ref[...] = g * sn_ref[...] + bn_ref[...]
    hp = g * sp_ref[...] + bp_ref[...]

    bn_rows = hp.shape[0]
    ngi = ngi_ref[0][0]
    onehot = (lax.broadcasted_iota(jnp.int32, (n_graphs, bn_rows), 0)
              == ngi[None, :]).astype(jnp.float32)
    contrib = jax.lax.dot(onehot, hp, precision=jax.lax.Precision.HIGHEST,
                          preferred_element_type=jnp.float32)

    @pl.when(i == 0)
    def _init():
        pool_ref[...] = contrib

    @pl.when(i != 0)
    def _accum():
        pool_ref[...] += contrib


def _make_tc_layer(n_nodes, d, h, n_graphs, bn_rows):
    assert n_nodes % bn_rows == 0
    grid = (n_nodes // bn_rows,)
    return pl.pallas_call(
        functools.partial(_tc_layer_body, n_graphs=n_graphs),
        grid=grid,
        in_specs=[
            pl.BlockSpec((bn_rows, d), lambda i: (i, 0)),
            pl.BlockSpec((NC, bn_rows, d), lambda i: (0, i, 0)),
            pl.BlockSpec((d, h), lambda i: (0, 0)),
            pl.BlockSpec((1, h), lambda i: (0, 0)),
            pl.BlockSpec((h, h), lambda i: (0, 0)),
            pl.BlockSpec((1, h), lambda i: (0, 0)),
            pl.BlockSpec((1, h), lambda i: (0, 0)),
            pl.BlockSpec((1, h), lambda i: (0, 0)),
            pl.BlockSpec((1, h), lambda i: (0, 0)),
            pl.BlockSpec((1, h), lambda i: (0, 0)),
            pl.BlockSpec((1, 1, bn_rows), lambda i: (i, 0, 0)),
        ],
        out_specs=[
            pl.BlockSpec((bn_rows, h), lambda i: (i, 0)),
            pl.BlockSpec((n_graphs, h), lambda i: (0, 0)),
        ],
        out_shape=[
            jax.ShapeDtypeStruct((n_nodes, h), jnp.float32),
            jax.ShapeDtypeStruct((n_graphs, h), jnp.float32),
        ],
        compiler_params=pltpu.CompilerParams(
            dimension_semantics=("arbitrary",)),
    )


def _mlp_body(p_ref, w1_ref, b1_ref, w2_ref, b2_ref, o_ref):
    nl = p_ref.shape[0]
    acc = b1_ref[...]
    for k in range(nl):
        acc = acc + jnp.dot(p_ref[k], w1_ref[k],
                            preferred_element_type=jnp.float32)
    hid = jnp.maximum(acc, 0.0)
    o_ref[...] = jnp.dot(hid, w2_ref[...],
                         preferred_element_type=jnp.float32) + b2_ref[...]


def kernel(x, edge_index, edge_weight, node_graph_index,
           gin_W1, gin_b1, gin_W2, gin_b2, bn_gamma, bn_beta,
           mlp_w1, mlp_b1, mlp_w2, mlp_b2):
    n_nodes, d = x.shape
    n_edges = edge_index.shape[1]
    h = gin_W2.shape[-1]
    n_graphs = 64
    n_cls = mlp_w2.shape[-1]
    bn_rows = 1000

    src = edge_index[0]
    dst = edge_index[1]

    inv = jnp.float32(1.0 / math.sqrt(1.0 + 1e-3))
    bn_s = bn_gamma * inv
    bn_b = bn_beta
    ones = jnp.ones((1, h), jnp.float32)
    zeros = jnp.zeros((1, h), jnp.float32)

    ngi3 = node_graph_index.reshape(n_nodes // bn_rows, 1, bn_rows)

    n_pad = -(-n_nodes // (NS * 128)) * (NS * 128)
    sc_aggregate = _make_sc_aggregate(n_pad, n_edges, d)
    tc_layer = _make_tc_layer(n_nodes, d, h, n_graphs, bn_rows)

    r = lambda v: v.reshape(1, h)
    cfg = [
        (ones, zeros, r(bn_s[0]), r(bn_b[0])),
        (ones, zeros, r(bn_s[0]), r(bn_b[0])),
        (ones, zeros, r(bn_s[1]), r(bn_b[1])),
        (ones, zeros, r(bn_s[2]), r(bn_b[2])),
        (r(bn_s[2]), r(bn_b[2]), zeros, zeros),
    ]

    t = x
    pools = []
    for i in range(5):
        aggr = sc_aggregate(t, src, dst, edge_weight)
        sp, bp, sn, bnx = cfg[i]
        t, pool = tc_layer(t, aggr, gin_W1[i], gin_b1[i].reshape(1, h),
                           gin_W2[i], gin_b2[i].reshape(1, h),
                           sp, bp, sn, bnx, ngi3)
        pools.append(pool)

    pstack = jnp.stack(pools, axis=0)
    w1r = mlp_w1.reshape(5, h, mlp_w1.shape[-1])
    hid_dim = mlp_w1.shape[-1]
    w2p = jnp.zeros((hid_dim, 128), jnp.float32).at[:, :n_cls].set(mlp_w2)
    b2p = jnp.zeros((1, 128), jnp.float32).at[0, :n_cls].set(mlp_b2)

    out_pad = pl.pallas_call(
        _mlp_body,
        out_shape=jax.ShapeDtypeStruct((n_graphs, 128), jnp.float32),
    )(pstack, w1r, mlp_b1.reshape(1, hid_dim), w2p, b2p)
    return out_pad[:, :n_cls]


# speedup vs baseline: 8.4114x; 2.1523x over previous
import functools
import math

import jax
import jax.numpy as jnp
from jax import lax
from jax.experimental import pallas as pl
from jax.experimental.pallas import tpu as pltpu
from jax.experimental.pallas import tpu_sc as plsc

NC = 2
NS = 16
NW = NC * NS


def _make_sc_aggregate(n_pad, n_edges, d):
    assert n_edges % NW == 0
    epw = n_edges // NW
    K = 80
    assert epw % K == 0
    n_chunks = epw // K
    RB = 128
    assert n_pad % (NS * RB) == 0
    rpt = n_pad // NS
    n_rb = rpt // RB
    nvr = d // 16
    assert n_chunks % 2 == 1

    mesh = plsc.VectorSubcoreMesh(
        core_axis_name="c", subcore_axis_name="s",
        num_cores=NC, num_subcores=NS)

    @functools.partial(
        pl.kernel,
        out_type=jax.ShapeDtypeStruct((NC, n_pad, d), jnp.float32),
        mesh=mesh,
        scratch_types=[
            pltpu.VMEM((2, 2, K), jnp.int32),
            pltpu.VMEM((2, K), jnp.float32),
            pltpu.VMEM((2, K, d), jnp.float32),
            pltpu.VMEM((RB, d), jnp.float32),
            pltpu.VMEM_SHARED((n_pad, d), jnp.float32),
            pltpu.SemaphoreType.DMA,
            pltpu.SemaphoreType.DMA,
            pltpu.SemaphoreType.DMA,
            pltpu.SemaphoreType.DMA,
        ],
    )
    def sc_aggregate(h_hbm, src_hbm, dst_hbm, w_hbm, out_hbm,
                     idx_v, w_v, rows_v, bounce_v, aggr_sh,
                     semg0, semg1, semi0, semi1):
        c = lax.axis_index("c")
        s = lax.axis_index("s")
        wid = c * NS + s
        semg = (semg0, semg1)
        semi = (semi0, semi1)
        ebase = wid * epw

        def issue_idx(ci, b):
            base = ebase + ci * K
            pltpu.async_copy(src_hbm.at[pl.ds(base, K)], idx_v.at[b, 0],
                             semi[b])
            pltpu.async_copy(dst_hbm.at[pl.ds(base, K)], idx_v.at[b, 1],
                             semi[b])
            pltpu.async_copy(w_hbm.at[pl.ds(base, K)], w_v.at[b], semi[b])

        def wait_idx(ci, b):
            base = ebase + ci * K
            pltpu.make_async_copy(src_hbm.at[pl.ds(base, K)], idx_v.at[b, 0],
                                  semi[b]).wait()
            pltpu.make_async_copy(dst_hbm.at[pl.ds(base, K)], idx_v.at[b, 1],
                                  semi[b]).wait()
            pltpu.make_async_copy(w_hbm.at[pl.ds(base, K)], w_v.at[b],
                                  semi[b]).wait()

        def issue_gather(ci, b):
            pltpu.async_copy(h_hbm.at[idx_v.at[b, 0]], rows_v.at[b], semg[b])

        def wait_gather(ci, b):
            pltpu.make_async_copy(h_hbm.at[idx_v.at[b, 0]], rows_v.at[b],
                                  semg[b]).wait()

        def process(ci, b):
            def group(gi, gcarry):
                wchunk = w_v[b, pl.ds(gi * 16, 16)]
                for j in range(16):
                    wb = lax.gather(
                        wchunk, jnp.full((16, 1), j, jnp.int32),
                        dimension_numbers=lax.GatherDimensionNumbers(
                            offset_dims=(), collapsed_slice_dims=(0,),
                            start_index_map=(0,)),
                        slice_sizes=(1,),
                        mode=lax.GatherScatterMode.PROMISE_IN_BOUNDS)
                    e = gi * 16 + j
                    for r in range(nvr):
                        sl = pl.ds(r * 16, 16)
                        rows_v[b, e, sl] = rows_v[b, e, sl] * wb
                return gcarry
            lax.fori_loop(0, K // 16, group, 0)
            pltpu.sync_copy(rows_v.at[b], aggr_sh.at[idx_v.at[b, 1]],
                            add=True)

        issue_idx(0, 0)
        issue_idx(1, 1)

        def zrow(i, carry):
            for r in range(nvr):
                bounce_v[i, pl.ds(r * 16, 16)] = jnp.zeros((16,), jnp.float32)
            return carry
        lax.fori_loop(0, RB, zrow, 0)
        row0 = s * rpt
        for b in range(n_rb):
            pltpu.sync_copy(bounce_v, aggr_sh.at[pl.ds(row0 + b * RB, RB)])
        plsc.subcore_barrier()

        wait_idx(0, 0)
        issue_gather(0, 0)

        def step(ci, b, issue_next):
            if issue_next:
                wait_idx(ci + 1, 1 - b)
                issue_gather(ci + 1, 1 - b)
            wait_gather(ci, b)
            process(ci, b)
            if issue_next:
                @pl.when(ci + 2 < n_chunks)
                def _():
                    issue_idx(ci + 2, b)

        def pair(g, carry):
            for b in range(2):
                step(g * 2 + b, b, True)
            return carry
        lax.fori_loop(0, (n_chunks - 1) // 2, pair, 0)
        step(n_chunks - 1, (n_chunks - 1) % 2, False)

        plsc.subcore_barrier()

        for b in range(n_rb):
            r0 = row0 + b * RB
            pltpu.sync_copy(aggr_sh.at[pl.ds(r0, RB)], bounce_v)
            pltpu.sync_copy(bounce_v, out_hbm.at[c, pl.ds(r0, RB)])

    return sc_aggregate


def _tc_layer_body(t_ref, a_ref, w1_ref, b1_ref, w2_ref, b2_ref,
                   sp_ref, bp_ref, sn_ref, bn_ref, ngi_ref,
                   tn_ref, pool_ref, *, n_graphs):
    i = pl.program_id(0)
    u = t_ref[...] + a_ref[0] + a_ref[1]
    z = jnp.dot(u, w1_ref[...], preferred_element_type=jnp.float32) + b1_ref[...]
    z = jnp.maximum(z, 0.0)
    g = jnp.dot(z, w2_ref[...], preferred_element_type=jnp.float32) + b2_ref[...]
    tn_ref[...] = g * sn_ref[...] + bn_ref[...]
    hp = g * sp_ref[...] + bp_ref[...]

    bn_rows = hp.shape[0]
    ngi = ngi_ref[0][0]
    onehot = (lax.broadcasted_iota(jnp.int32, (n_graphs, bn_rows), 0)
              == ngi[None, :]).astype(jnp.float32)
    contrib = jax.lax.dot(onehot, hp, precision=jax.lax.Precision.HIGHEST,
                          preferred_element_type=jnp.float32)

    @pl.when(i == 0)
    def _init():
        pool_ref[...] = contrib

    @pl.when(i != 0)
    def _accum():
        pool_ref[...] += contrib


def _make_tc_layer(n_nodes, d, h, n_graphs, bn_rows):
    assert n_nodes % bn_rows == 0
    grid = (n_nodes // bn_rows,)
    return pl.pallas_call(
        functools.partial(_tc_layer_body, n_graphs=n_graphs),
        grid=grid,
        in_specs=[
            pl.BlockSpec((bn_rows, d), lambda i: (i, 0)),
            pl.BlockSpec((NC, bn_rows, d), lambda i: (0, i, 0)),
            pl.BlockSpec((d, h), lambda i: (0, 0)),
            pl.BlockSpec((1, h), lambda i: (0, 0)),
            pl.BlockSpec((h, h), lambda i: (0, 0)),
            pl.BlockSpec((1, h), lambda i: (0, 0)),
            pl.BlockSpec((1, h), lambda i: (0, 0)),
            pl.BlockSpec((1, h), lambda i: (0, 0)),
            pl.BlockSpec((1, h), lambda i: (0, 0)),
            pl.BlockSpec((1, h), lambda i: (0, 0)),
            pl.BlockSpec((1, 1, bn_rows), lambda i: (i, 0, 0)),
        ],
        out_specs=[
            pl.BlockSpec((bn_rows, h), lambda i: (i, 0)),
            pl.BlockSpec((n_graphs, h), lambda i: (0, 0)),
        ],
        out_shape=[
            jax.ShapeDtypeStruct((n_nodes, h), jnp.float32),
            jax.ShapeDtypeStruct((n_graphs, h), jnp.float32),
        ],
        compiler_params=pltpu.CompilerParams(
            dimension_semantics=("arbitrary",)),
    )


def _mlp_body(p_ref, w1_ref, b1_ref, w2_ref, b2_ref, o_ref):
    nl = p_ref.shape[0]
    acc = b1_ref[...]
    for k in range(nl):
        acc = acc + jnp.dot(p_ref[k], w1_ref[k],
                            preferred_element_type=jnp.float32)
    hid = jnp.maximum(acc, 0.0)
    o_ref[...] = jnp.dot(hid, w2_ref[...],
                         preferred_element_type=jnp.float32) + b2_ref[...]


def kernel(x, edge_index, edge_weight, node_graph_index,
           gin_W1, gin_b1, gin_W2, gin_b2, bn_gamma, bn_beta,
           mlp_w1, mlp_b1, mlp_w2, mlp_b2):
    n_nodes, d = x.shape
    n_edges = edge_index.shape[1]
    h = gin_W2.shape[-1]
    n_graphs = 64
    n_cls = mlp_w2.shape[-1]
    bn_rows = 1000

    src = edge_index[0]
    dst = edge_index[1]

    inv = jnp.float32(1.0 / math.sqrt(1.0 + 1e-3))
    bn_s = bn_gamma * inv
    bn_b = bn_beta
    ones = jnp.ones((1, h), jnp.float32)
    zeros = jnp.zeros((1, h), jnp.float32)

    ngi3 = node_graph_index.reshape(n_nodes // bn_rows, 1, bn_rows)

    n_pad = -(-n_nodes // (NS * 128)) * (NS * 128)
    sc_aggregate = _make_sc_aggregate(n_pad, n_edges, d)
    tc_layer = _make_tc_layer(n_nodes, d, h, n_graphs, bn_rows)

    r = lambda v: v.reshape(1, h)
    cfg = [
        (ones, zeros, r(bn_s[0]), r(bn_b[0])),
        (ones, zeros, r(bn_s[0]), r(bn_b[0])),
        (ones, zeros, r(bn_s[1]), r(bn_b[1])),
        (ones, zeros, r(bn_s[2]), r(bn_b[2])),
        (r(bn_s[2]), r(bn_b[2]), zeros, zeros),
    ]

    t = x
    pools = []
    for i in range(5):
        aggr = sc_aggregate(t, src, dst, edge_weight)
        sp, bp, sn, bnx = cfg[i]
        t, pool = tc_layer(t, aggr, gin_W1[i], gin_b1[i].reshape(1, h),
                           gin_W2[i], gin_b2[i].reshape(1, h),
                           sp, bp, sn, bnx, ngi3)
        pools.append(pool)

    pstack = jnp.stack(pools, axis=0)
    w1r = mlp_w1.reshape(5, h, mlp_w1.shape[-1])
    hid_dim = mlp_w1.shape[-1]
    w2p = jnp.zeros((hid_dim, 128), jnp.float32).at[:, :n_cls].set(mlp_w2)
    b2p = jnp.zeros((1, 128), jnp.float32).at[0, :n_cls].set(mlp_b2)

    out_pad = pl.pallas_call(
        _mlp_body,
        out_shape=jax.ShapeDtypeStruct((n_graphs, 128), jnp.float32),
    )(pstack, w1r, mlp_b1.reshape(1, hid_dim), w2p, b2p)
    return out_pad[:, :n_cls]


# speedup vs baseline: 10.1788x; 1.2101x over previous
import functools
import math

import jax
import jax.numpy as jnp
from jax import lax
from jax.experimental import pallas as pl
from jax.experimental.pallas import tpu as pltpu
from jax.experimental.pallas import tpu_sc as plsc

NC = 2
NS = 16
NW = NC * NS


def _make_sc_aggregate(n_pad, n_edges, d):
    assert n_edges % NW == 0
    epw = n_edges // NW
    K = 80
    assert epw % K == 0
    n_chunks = epw // K
    RB = 128
    assert n_pad % (NS * RB) == 0
    rpt = n_pad // NS
    n_rb = rpt // RB
    nvr = d // 16
    assert n_chunks % 2 == 1

    mesh = plsc.VectorSubcoreMesh(
        core_axis_name="c", subcore_axis_name="s",
        num_cores=NC, num_subcores=NS)

    @functools.partial(
        pl.kernel,
        out_type=jax.ShapeDtypeStruct((NC, n_pad, d), jnp.float32),
        mesh=mesh,
        scratch_types=[
            pltpu.VMEM((2, 2, K), jnp.int32),
            pltpu.VMEM((2, K), jnp.float32),
            pltpu.VMEM((2, K, d), jnp.float32),
            pltpu.VMEM((RB, d), jnp.float32),
            pltpu.VMEM_SHARED((n_pad, d), jnp.float32),
            pltpu.SemaphoreType.DMA,
            pltpu.SemaphoreType.DMA,
            pltpu.SemaphoreType.DMA,
            pltpu.SemaphoreType.DMA,
            pltpu.SemaphoreType.DMA,
            pltpu.SemaphoreType.DMA,
        ],
    )
    def sc_aggregate(h_hbm, src_hbm, dst_hbm, w_hbm, out_hbm,
                     idx_v, w_v, rows_v, bounce_v, aggr_sh,
                     semg0, semg1, semi0, semi1, sems0, sems1):
        c = lax.axis_index("c")
        s = lax.axis_index("s")
        wid = c * NS + s
        semg = (semg0, semg1)
        semi = (semi0, semi1)
        sems = (sems0, sems1)
        ebase = wid * epw

        def issue_idx(ci, b):
            base = ebase + ci * K
            pltpu.async_copy(src_hbm.at[pl.ds(base, K)], idx_v.at[b, 0],
                             semi[b])
            pltpu.async_copy(dst_hbm.at[pl.ds(base, K)], idx_v.at[b, 1],
                             semi[b])
            pltpu.async_copy(w_hbm.at[pl.ds(base, K)], w_v.at[b], semi[b])

        def wait_idx(ci, b):
            base = ebase + ci * K
            pltpu.make_async_copy(src_hbm.at[pl.ds(base, K)], idx_v.at[b, 0],
                                  semi[b]).wait()
            pltpu.make_async_copy(dst_hbm.at[pl.ds(base, K)], idx_v.at[b, 1],
                                  semi[b]).wait()
            pltpu.make_async_copy(w_hbm.at[pl.ds(base, K)], w_v.at[b],
                                  semi[b]).wait()

        def issue_gather(ci, b):
            pltpu.async_copy(h_hbm.at[idx_v.at[b, 0]], rows_v.at[b], semg[b])

        def wait_gather(ci, b):
            pltpu.make_async_copy(h_hbm.at[idx_v.at[b, 0]], rows_v.at[b],
                                  semg[b]).wait()

        def process(ci, b):
            def group(gi, gcarry):
                wchunk = w_v[b, pl.ds(gi * 16, 16)]
                for j in range(16):
                    wb = lax.gather(
                        wchunk, jnp.full((16, 1), j, jnp.int32),
                        dimension_numbers=lax.GatherDimensionNumbers(
                            offset_dims=(), collapsed_slice_dims=(0,),
                            start_index_map=(0,)),
                        slice_sizes=(1,),
                        mode=lax.GatherScatterMode.PROMISE_IN_BOUNDS)
                    e = gi * 16 + j
                    for r in range(nvr):
                        sl = pl.ds(r * 16, 16)
                        rows_v[b, e, sl] = rows_v[b, e, sl] * wb
                return gcarry
            lax.fori_loop(0, K // 16, group, 0)
            pltpu.async_copy(rows_v.at[b], aggr_sh.at[idx_v.at[b, 1]],
                             sems[b], add=True)

        def wait_scatter(b):
            pltpu.make_async_copy(rows_v.at[b], aggr_sh.at[idx_v.at[b, 1]],
                                  sems[b]).wait()

        issue_idx(0, 0)
        issue_idx(1, 1)

        def zrow(i, carry):
            for r in range(nvr):
                bounce_v[i, pl.ds(r * 16, 16)] = jnp.zeros((16,), jnp.float32)
            return carry
        lax.fori_loop(0, RB, zrow, 0)
        row0 = s * rpt
        for b in range(n_rb):
            pltpu.sync_copy(bounce_v, aggr_sh.at[pl.ds(row0 + b * RB, RB)])
        plsc.subcore_barrier()

        wait_idx(0, 0)
        issue_gather(0, 0)

        def step(ci, b, issue_next):
            if issue_next:
                wait_idx(ci + 1, 1 - b)

            @pl.when(ci > 0)
            def _():
                wait_scatter(1 - b)

            if issue_next:
                issue_gather(ci + 1, 1 - b)
            wait_gather(ci, b)
            process(ci, b)
            if issue_next:
                @pl.when(ci + 2 < n_chunks)
                def _():
                    issue_idx(ci + 2, b)

        def pair(g, carry):
            for b in range(2):
                step(g * 2 + b, b, True)
            return carry
        lax.fori_loop(0, (n_chunks - 1) // 2, pair, 0)
        last = n_chunks - 1
        step(last, last % 2, False)
        wait_scatter(last % 2)

        plsc.subcore_barrier()

        for b in range(n_rb):
            r0 = row0 + b * RB
            pltpu.sync_copy(aggr_sh.at[pl.ds(r0, RB)], bounce_v)
            pltpu.sync_copy(bounce_v, out_hbm.at[c, pl.ds(r0, RB)])

    return sc_aggregate


def _tc_layer_body(t_ref, a_ref, w1_ref, b1_ref, w2_ref, b2_ref,
                   sp_ref, bp_ref, sn_ref, bn_ref, ngi_ref,
                   tn_ref, pool_ref, *, n_graphs):
    i = pl.program_id(0)
    u = t_ref[...] + a_ref[0] + a_ref[1]
    z = jnp.dot(u, w1_ref[...], preferred_element_type=jnp.float32) + b1_ref[...]
    z = jnp.maximum(z, 0.0)
    g = jnp.dot(z, w2_ref[...], preferred_element_type=jnp.float32) + b2_ref[...]
    tn_ref[...] = g * sn_ref[...] + bn_ref[...]
    hp = g * sp_ref[...] + bp_ref[...]

    bn_rows = hp.shape[0]
    ngi = ngi_ref[0][0]
    onehot = (lax.broadcasted_iota(jnp.int32, (n_graphs, bn_rows), 0)
              == ngi[None, :]).astype(jnp.float32)
    contrib = jax.lax.dot(onehot, hp, precision=jax.lax.Precision.HIGHEST,
                          preferred_element_type=jnp.float32)

    @pl.when(i == 0)
    def _init():
        pool_ref[...] = contrib

    @pl.when(i != 0)
    def _accum():
        pool_ref[...] += contrib


def _make_tc_layer(n_nodes, d, h, n_graphs, bn_rows):
    assert n_nodes % bn_rows == 0
    grid = (n_nodes // bn_rows,)
    return pl.pallas_call(
        functools.partial(_tc_layer_body, n_graphs=n_graphs),
        grid=grid,
        in_specs=[
            pl.BlockSpec((bn_rows, d), lambda i: (i, 0)),
            pl.BlockSpec((NC, bn_rows, d), lambda i: (0, i, 0)),
            pl.BlockSpec((d, h), lambda i: (0, 0)),
            pl.BlockSpec((1, h), lambda i: (0, 0)),
            pl.BlockSpec((h, h), lambda i: (0, 0)),
            pl.BlockSpec((1, h), lambda i: (0, 0)),
            pl.BlockSpec((1, h), lambda i: (0, 0)),
            pl.BlockSpec((1, h), lambda i: (0, 0)),
            pl.BlockSpec((1, h), lambda i: (0, 0)),
            pl.BlockSpec((1, h), lambda i: (0, 0)),
            pl.BlockSpec((1, 1, bn_rows), lambda i: (i, 0, 0)),
        ],
        out_specs=[
            pl.BlockSpec((bn_rows, h), lambda i: (i, 0)),
            pl.BlockSpec((n_graphs, h), lambda i: (0, 0)),
        ],
        out_shape=[
            jax.ShapeDtypeStruct((n_nodes, h), jnp.float32),
            jax.ShapeDtypeStruct((n_graphs, h), jnp.float32),
        ],
        compiler_params=pltpu.CompilerParams(
            dimension_semantics=("arbitrary",)),
    )


def _mlp_body(p_ref, w1_ref, b1_ref, w2_ref, b2_ref, o_ref):
    nl = p_ref.shape[0]
    acc = b1_ref[...]
    for k in range(nl):
        acc = acc + jnp.dot(p_ref[k], w1_ref[k],
                            preferred_element_type=jnp.float32)
    hid = jnp.maximum(acc, 0.0)
    o_ref[...] = jnp.dot(hid, w2_ref[...],
                         preferred_element_type=jnp.float32) + b2_ref[...]


def kernel(x, edge_index, edge_weight, node_graph_index,
           gin_W1, gin_b1, gin_W2, gin_b2, bn_gamma, bn_beta,
           mlp_w1, mlp_b1, mlp_w2, mlp_b2):
    n_nodes, d = x.shape
    n_edges = edge_index.shape[1]
    h = gin_W2.shape[-1]
    n_graphs = 64
    n_cls = mlp_w2.shape[-1]
    bn_rows = 1000

    src = edge_index[0]
    dst = edge_index[1]

    inv = jnp.float32(1.0 / math.sqrt(1.0 + 1e-3))
    bn_s = bn_gamma * inv
    bn_b = bn_beta
    ones = jnp.ones((1, h), jnp.float32)
    zeros = jnp.zeros((1, h), jnp.float32)

    ngi3 = node_graph_index.reshape(n_nodes // bn_rows, 1, bn_rows)

    n_pad = -(-n_nodes // (NS * 128)) * (NS * 128)
    sc_aggregate = _make_sc_aggregate(n_pad, n_edges, d)
    tc_layer = _make_tc_layer(n_nodes, d, h, n_graphs, bn_rows)

    r = lambda v: v.reshape(1, h)
    cfg = [
        (ones, zeros, r(bn_s[0]), r(bn_b[0])),
        (ones, zeros, r(bn_s[0]), r(bn_b[0])),
        (ones, zeros, r(bn_s[1]), r(bn_b[1])),
        (ones, zeros, r(bn_s[2]), r(bn_b[2])),
        (r(bn_s[2]), r(bn_b[2]), zeros, zeros),
    ]

    t = x
    pools = []
    for i in range(5):
        aggr = sc_aggregate(t, src, dst, edge_weight)
        sp, bp, sn, bnx = cfg[i]
        t, pool = tc_layer(t, aggr, gin_W1[i], gin_b1[i].reshape(1, h),
                           gin_W2[i], gin_b2[i].reshape(1, h),
                           sp, bp, sn, bnx, ngi3)
        pools.append(pool)

    pstack = jnp.stack(pools, axis=0)
    w1r = mlp_w1.reshape(5, h, mlp_w1.shape[-1])
    hid_dim = mlp_w1.shape[-1]
    w2p = jnp.zeros((hid_dim, 128), jnp.float32).at[:, :n_cls].set(mlp_w2)
    b2p = jnp.zeros((1, 128), jnp.float32).at[0, :n_cls].set(mlp_b2)

    out_pad = pl.pallas_call(
        _mlp_body,
        out_shape=jax.ShapeDtypeStruct((n_graphs, 128), jnp.float32),
    )(pstack, w1r, mlp_b1.reshape(1, hid_dim), w2p, b2p)
    return out_pad[:, :n_cls]
